# baseline (speedup 1.0000x reference)
import functools

import jax
import jax.numpy as jnp
from jax import lax
from jax.experimental import pallas as pl
from jax.experimental.pallas import tpu as pltpu
from jax.experimental.pallas import tpu_sc as plsc

_B, _P, _D = 64, 1024, 96
_NW = 32
_ROWS = _P // _NW
_LANES = 16


def _sc_body(x_hbm, p_hbm, o_hbm, pos_v, xbuf, psem, isems, osems):
    wid = lax.axis_index("s") * 2 + lax.axis_index("c")
    p0 = wid * _ROWS

    pltpu.async_copy(p_hbm.at[pl.ds(p0, _ROWS)], pos_v, psem).wait()

    def in_cp(b, s):
        return pltpu.make_async_copy(
            x_hbm.at[b, pl.ds(p0, _ROWS)], xbuf.at[s], isems.at[s])

    def out_cp(b, s):
        return pltpu.make_async_copy(
            xbuf.at[s], o_hbm.at[b, pl.ds(p0, _ROWS)], osems.at[s])

    in_cp(0, 0).start()
    in_cp(1, 1).start()

    def step(b, carry):
        s = lax.rem(b, 2)
        in_cp(b, s).wait()

        def add_rows(r, c2):
            for j in range(_D // _LANES):
                sl = pl.ds(j * _LANES, _LANES)
                xbuf[s, r, sl] = xbuf[s, r, sl] + pos_v[r, sl]
            return c2

        lax.fori_loop(0, _ROWS, add_rows, 0, unroll=4)
        out_cp(b, s).start()

        @pl.when(b + 2 < _B)
        def _():
            out_cp(b, s).wait()
            in_cp(b + 2, s).start()

        return carry

    lax.fori_loop(0, _B, step, 0)
    out_cp(_B - 2, 0).wait()
    out_cp(_B - 1, 1).wait()


def kernel(x, pos_table):
    B, P, D = x.shape
    mesh = plsc.VectorSubcoreMesh(core_axis_name="c", subcore_axis_name="s")
    run = pl.kernel(
        _sc_body,
        mesh=mesh,
        out_type=jax.ShapeDtypeStruct((B, P, D), jnp.float32),
        scratch_types=[
            pltpu.VMEM((_ROWS, _D), jnp.float32),
            pltpu.VMEM((2, _ROWS, _D), jnp.float32),
            pltpu.SemaphoreType.DMA,
            pltpu.SemaphoreType.DMA((2,)),
            pltpu.SemaphoreType.DMA((2,)),
        ],
    )
    return run(x, pos_table)


# speedup vs baseline: 2.4088x; 2.4088x over previous
import jax
import jax.numpy as jnp
from jax import lax
from jax.experimental import pallas as pl
from jax.experimental.pallas import tpu as pltpu
from jax.experimental.pallas import tpu_sc as plsc

_B, _D, _P = 64, 96, 1024
_NW = 32
_DG = 8
_NG = _D // _DG
_NS = 2
_LANES = 16
_SLICES = _DG * _P // _LANES


def _sc_body(x_hbm, p_hbm, o_hbm, pos_v, ibuf, obuf, psems, isems, osems):
    wid = lax.axis_index("s") * 2 + lax.axis_index("c")
    b0 = wid * 2

    def pos_cp(g, s):
        return pltpu.make_async_copy(
            p_hbm.at[pl.ds(g * _DG, _DG)], pos_v.at[s], psems.at[s])

    def in_cp(g, s):
        return pltpu.make_async_copy(
            x_hbm.at[pl.ds(b0, 2), pl.ds(g * _DG, _DG)], ibuf.at[s],
            isems.at[s])

    def out_cp(g, s):
        return pltpu.make_async_copy(
            obuf.at[s], o_hbm.at[pl.ds(b0, 2), pl.ds(g * _DG, _DG)],
            osems.at[s])

    for s in range(_NS):
        pos_cp(s, s).start()
        in_cp(s, s).start()

    for g in range(_NG):
        s = g % _NS
        in_cp(g, s).wait()
        pos_cp(g, s).wait()
        if g >= _NS:
            out_cp(g - _NS, s).wait()

        def add_slices(j, carry, s=s):
            r = j // (_P // _LANES)
            col = (j % (_P // _LANES)) * _LANES
            sl = pl.ds(col, _LANES)
            pv = pos_v[s, r, sl]
            obuf[s, 0, r, sl] = ibuf[s, 0, r, sl] + pv
            obuf[s, 1, r, sl] = ibuf[s, 1, r, sl] + pv
            return carry

        lax.fori_loop(0, _SLICES, add_slices, 0, unroll=2)
        out_cp(g, s).start()
        if g + _NS < _NG:
            in_cp(g + _NS, s).start()
            pos_cp(g + _NS, s).start()

    for g in range(_NG - _NS, _NG):
        out_cp(g, g % _NS).wait()


def kernel(x, pos_table):
    xt = jnp.swapaxes(x, 1, 2)
    pt = jnp.swapaxes(pos_table, 0, 1)
    mesh = plsc.VectorSubcoreMesh(core_axis_name="c", subcore_axis_name="s")
    run = pl.kernel(
        _sc_body,
        mesh=mesh,
        out_type=jax.ShapeDtypeStruct((_B, _D, _P), jnp.float32),
        scratch_types=[
            pltpu.VMEM((_NS, _DG, _P), jnp.float32),
            pltpu.VMEM((_NS, 2, _DG, _P), jnp.float32),
            pltpu.VMEM((_NS, 2, _DG, _P), jnp.float32),
            pltpu.SemaphoreType.DMA((_NS,)),
            pltpu.SemaphoreType.DMA((_NS,)),
            pltpu.SemaphoreType.DMA((_NS,)),
        ],
    )
    out = run(xt, pt)
    return jnp.swapaxes(out, 1, 2)


# speedup vs baseline: 2.5322x; 1.0513x over previous
import jax
import jax.numpy as jnp
from jax import lax
from jax.experimental import pallas as pl
from jax.experimental.pallas import tpu as pltpu
from jax.experimental.pallas import tpu_sc as plsc

_B, _D, _P = 64, 96, 1024
_NW = 32
_DG = 8
_NG = _D // _DG
_NS = 3
_LANES = 16
_SLICES = _DG * _P // _LANES


def _sc_body(x_hbm, p_hbm, o_hbm, pos_v, ibuf, obuf, psems, isems, osems):
    wid = lax.axis_index("s") * 2 + lax.axis_index("c")
    b0 = wid * 2

    def pos_cp(g, s):
        return pltpu.make_async_copy(
            p_hbm.at[pl.ds(g * _DG, _DG)], pos_v.at[s], psems.at[s])

    def in_cp(g, s):
        return pltpu.make_async_copy(
            x_hbm.at[pl.ds(b0, 2), pl.ds(g * _DG, _DG)], ibuf.at[s],
            isems.at[s])

    def out_cp(g, s):
        return pltpu.make_async_copy(
            obuf.at[s], o_hbm.at[pl.ds(b0, 2), pl.ds(g * _DG, _DG)],
            osems.at[s])

    for s in range(_NS):
        pos_cp(s, s).start()
        in_cp(s, s).start()

    for g in range(_NG):
        s = g % _NS
        in_cp(g, s).wait()
        pos_cp(g, s).wait()
        if g >= _NS:
            out_cp(g - _NS, s).wait()

        def add_slices(j, carry, s=s):
            r = j // (_P // _LANES)
            col = (j % (_P // _LANES)) * _LANES
            sl = pl.ds(col, _LANES)
            pv = pos_v[s, r, sl]
            obuf[s, 0, r, sl] = ibuf[s, 0, r, sl] + pv
            obuf[s, 1, r, sl] = ibuf[s, 1, r, sl] + pv
            return carry

        lax.fori_loop(0, _SLICES, add_slices, 0, unroll=8)
        out_cp(g, s).start()
        if g + _NS < _NG:
            in_cp(g + _NS, s).start()
            pos_cp(g + _NS, s).start()

    for g in range(_NG - _NS, _NG):
        out_cp(g, g % _NS).wait()


def kernel(x, pos_table):
    xt = jnp.swapaxes(x, 1, 2)
    pt = jnp.swapaxes(pos_table, 0, 1)
    mesh = plsc.VectorSubcoreMesh(core_axis_name="c", subcore_axis_name="s")
    run = pl.kernel(
        _sc_body,
        mesh=mesh,
        out_type=jax.ShapeDtypeStruct((_B, _D, _P), jnp.float32),
        scratch_types=[
            pltpu.VMEM((_NS, _DG, _P), jnp.float32),
            pltpu.VMEM((_NS, 2, _DG, _P), jnp.float32),
            pltpu.VMEM((_NS, 2, _DG, _P), jnp.float32),
            pltpu.SemaphoreType.DMA((_NS,)),
            pltpu.SemaphoreType.DMA((_NS,)),
            pltpu.SemaphoreType.DMA((_NS,)),
        ],
    )
    out = run(xt, pt)
    return jnp.swapaxes(out, 1, 2)


# speedup vs baseline: 8.3008x; 3.2781x over previous
import jax
import jax.numpy as jnp
from jax.experimental import pallas as pl
from jax.experimental.pallas import tpu as pltpu

_B, _D, _P = 64, 96, 1024
_CB = 2
_NCH = _B // _CB
_NBUF = 6


def _add_body(x_hbm, p_ref, o_hbm, ibuf, obuf, isems, osems):
    pos = p_ref[...]

    def in_cp(c, s):
        return pltpu.make_async_copy(
            x_hbm.at[pl.ds(c * _CB, _CB)], ibuf.at[s], isems.at[s])

    def out_cp(c, s):
        return pltpu.make_async_copy(
            obuf.at[s], o_hbm.at[pl.ds(c * _CB, _CB)], osems.at[s])

    for s in range(_NBUF):
        in_cp(s, s).start()
    for c in range(_NCH):
        s = c % _NBUF
        in_cp(c, s).wait()
        if c >= _NBUF:
            out_cp(c - _NBUF, s).wait()
        obuf[s] = ibuf[s] + pos
        out_cp(c, s).start()
        if c + _NBUF < _NCH:
            in_cp(c + _NBUF, s).start()
    for c in range(_NCH - _NBUF, _NCH):
        out_cp(c, c % _NBUF).wait()


def kernel(x, pos_table):
    xt = jnp.swapaxes(x, 1, 2)
    pt = jnp.swapaxes(pos_table, 0, 1)
    out = pl.pallas_call(
        _add_body,
        in_specs=[
            pl.BlockSpec(memory_space=pl.ANY),
            pl.BlockSpec(memory_space=pltpu.MemorySpace.VMEM),
        ],
        out_specs=pl.BlockSpec(memory_space=pl.ANY),
        out_shape=jax.ShapeDtypeStruct((_B, _D, _P), jnp.float32),
        scratch_shapes=[
            pltpu.VMEM((_NBUF, _CB, _D, _P), jnp.float32),
            pltpu.VMEM((_NBUF, _CB, _D, _P), jnp.float32),
            pltpu.SemaphoreType.DMA((_NBUF,)),
            pltpu.SemaphoreType.DMA((_NBUF,)),
        ],
    )(xt, pt)
    return jnp.swapaxes(out, 1, 2)


# speedup vs baseline: 8.5071x; 1.0249x over previous
import jax
import jax.numpy as jnp
from jax.experimental import pallas as pl
from jax.experimental.pallas import tpu as pltpu

_B, _D, _P = 64, 96, 1024
_CB = 4
_NCH = _B // _CB
_NBUF = 4


def _add_body(x_hbm, p_ref, o_hbm, ibuf, obuf, isems, osems):
    pos = p_ref[...]

    def in_cp(c, s):
        return pltpu.make_async_copy(
            x_hbm.at[pl.ds(c * _CB, _CB)], ibuf.at[s], isems.at[s])

    def out_cp(c, s):
        return pltpu.make_async_copy(
            obuf.at[s], o_hbm.at[pl.ds(c * _CB, _CB)], osems.at[s])

    for s in range(_NBUF):
        in_cp(s, s).start()
    for c in range(_NCH):
        s = c % _NBUF
        in_cp(c, s).wait()
        if c >= _NBUF:
            out_cp(c - _NBUF, s).wait()
        obuf[s] = ibuf[s] + pos
        out_cp(c, s).start()
        if c + _NBUF < _NCH:
            in_cp(c + _NBUF, s).start()
    for c in range(_NCH - _NBUF, _NCH):
        out_cp(c, c % _NBUF).wait()


def kernel(x, pos_table):
    xt = jnp.swapaxes(x, 1, 2)
    pt = jnp.swapaxes(pos_table, 0, 1)
    out = pl.pallas_call(
        _add_body,
        in_specs=[
            pl.BlockSpec(memory_space=pl.ANY),
            pl.BlockSpec(memory_space=pltpu.MemorySpace.VMEM),
        ],
        out_specs=pl.BlockSpec(memory_space=pl.ANY),
        out_shape=jax.ShapeDtypeStruct((_B, _D, _P), jnp.float32),
        scratch_shapes=[
            pltpu.VMEM((_NBUF, _CB, _D, _P), jnp.float32),
            pltpu.VMEM((_NBUF, _CB, _D, _P), jnp.float32),
            pltpu.SemaphoreType.DMA((_NBUF,)),
            pltpu.SemaphoreType.DMA((_NBUF,)),
        ],
    )(xt, pt)
    return jnp.swapaxes(out, 1, 2)


# speedup vs baseline: 8.6771x; 1.0200x over previous
import jax
import jax.numpy as jnp
from jax.experimental import pallas as pl
from jax.experimental.pallas import tpu as pltpu

_B, _D, _P = 64, 96, 1024
_CB = 8
_NCH = _B // _CB
_NBUF = 4


def _add_body(x_hbm, p_ref, o_hbm, ibuf, obuf, isems, osems):
    pos = p_ref[...]

    def in_cp(c, s):
        return pltpu.make_async_copy(
            x_hbm.at[pl.ds(c * _CB, _CB)], ibuf.at[s], isems.at[s])

    def out_cp(c, s):
        return pltpu.make_async_copy(
            obuf.at[s], o_hbm.at[pl.ds(c * _CB, _CB)], osems.at[s])

    for s in range(_NBUF):
        in_cp(s, s).start()
    for c in range(_NCH):
        s = c % _NBUF
        in_cp(c, s).wait()
        if c >= _NBUF:
            out_cp(c - _NBUF, s).wait()
        obuf[s] = ibuf[s] + pos
        out_cp(c, s).start()
        if c + _NBUF < _NCH:
            in_cp(c + _NBUF, s).start()
    for c in range(_NCH - _NBUF, _NCH):
        out_cp(c, c % _NBUF).wait()


def kernel(x, pos_table):
    xt = jnp.swapaxes(x, 1, 2)
    pt = jnp.swapaxes(pos_table, 0, 1)
    out = pl.pallas_call(
        _add_body,
        in_specs=[
            pl.BlockSpec(memory_space=pl.ANY),
            pl.BlockSpec(memory_space=pltpu.MemorySpace.VMEM),
        ],
        out_specs=pl.BlockSpec(memory_space=pl.ANY),
        out_shape=jax.ShapeDtypeStruct((_B, _D, _P), jnp.float32),
        scratch_shapes=[
            pltpu.VMEM((_NBUF, _CB, _D, _P), jnp.float32),
            pltpu.VMEM((_NBUF, _CB, _D, _P), jnp.float32),
            pltpu.SemaphoreType.DMA((_NBUF,)),
            pltpu.SemaphoreType.DMA((_NBUF,)),
        ],
    )(xt, pt)
    return jnp.swapaxes(out, 1, 2)
